# baseline (speedup 1.0000x reference)
import functools

import jax
import jax.numpy as jnp
from jax import lax
from jax.experimental import pallas as pl
from jax.experimental.pallas import tpu as pltpu
from jax.experimental.pallas import tpu_sc as plsc

N = 10000
DEG = 16
D = 256
L = 16
C = 8
NW = 32
NCHUNKS = N // C
_BASE_CHUNKS = NCHUNKS // NW
_EXTRA = NCHUNKS % NW

_MESH = plsc.VectorSubcoreMesh(core_axis_name="c", subcore_axis_name="s")


@functools.partial(
    pl.kernel,
    mesh=_MESH,
    out_type=jax.ShapeDtypeStruct((N, D), jnp.float32),
    scratch_types=[
        pltpu.VMEM((C, DEG, D), jnp.float32),
        pltpu.VMEM((C, 1, D), jnp.float32),
        pltpu.VMEM((C, DEG), jnp.float32),
        pltpu.VMEM((C, D), jnp.float32),
    ],
)
def _sc_kernel(curr_hbm, alpha_hbm, msg_hbm, out_hbm, msg_v, curr_v, alpha_v, out_v):
    wid = lax.axis_index("s") * 2 + lax.axis_index("c")
    nch = _BASE_CHUNKS + jnp.where(wid < _EXTRA, 1, 0)

    def chunk_body(i, _):
        base = (wid + i * NW) * C
        pltpu.sync_copy(msg_hbm.at[pl.ds(base, C)], msg_v)
        pltpu.sync_copy(curr_hbm.at[pl.ds(base, C), pl.ds(0, 1)], curr_v)
        pltpu.sync_copy(alpha_hbm.at[pl.ds(base, C)], alpha_v)

        def node_body(n, _):
            av = alpha_v[n, pl.ds(0, DEG)]
            a = [av[k] for k in range(DEG)]
            for j in range(D // L):
                acc = curr_v[n, 0, pl.ds(j * L, L)]
                for k in range(DEG):
                    acc = acc + a[k] * msg_v[n, k, pl.ds(j * L, L)]
                out_v[n, pl.ds(j * L, L)] = acc
            return 0

        lax.fori_loop(0, C, node_body, 0)
        pltpu.sync_copy(out_v, out_hbm.at[pl.ds(base, C)])
        return 0

    lax.fori_loop(0, nch, chunk_body, 0)


def kernel(curr_emb, alpha, msg):
    alpha2 = alpha.reshape(N, DEG)
    return _sc_kernel(curr_emb, alpha2, msg)


# speedup vs baseline: 1.8170x; 1.8170x over previous
import functools

import jax
import jax.numpy as jnp
from jax import lax
from jax.experimental import pallas as pl
from jax.experimental.pallas import tpu as pltpu
from jax.experimental.pallas import tpu_sc as plsc

N = 10000
DEG = 16
D = 256
L = 16
C = 8
NW = 32
NCHUNKS = N // C
STEADY = NCHUNKS // NW
_EXTRA = NCHUNKS % NW

_MESH = plsc.VectorSubcoreMesh(core_axis_name="c", subcore_axis_name="s")


@functools.partial(
    pl.kernel,
    mesh=_MESH,
    out_type=jax.ShapeDtypeStruct((N, D), jnp.float32),
    scratch_types=[
        pltpu.VMEM((2, C, DEG, D), jnp.float32),
        pltpu.VMEM((2, C, 1, D), jnp.float32),
        pltpu.VMEM((2, C, DEG), jnp.float32),
        pltpu.VMEM((2, C, D), jnp.float32),
        pltpu.SemaphoreType.DMA,
        pltpu.SemaphoreType.DMA,
        pltpu.SemaphoreType.DMA,
        pltpu.SemaphoreType.DMA,
    ],
)
def _sc_kernel(curr_hbm, alpha_hbm, msg_hbm, out_hbm,
               msg_v, curr_v, alpha_v, out_v, sin0, sin1, sout0, sout1):
    wid = lax.axis_index("s") * 2 + lax.axis_index("c")
    sin = (sin0, sin1)
    sout = (sout0, sout1)

    def in_copies(i, slot):
        base = (wid + i * NW) * C
        return (
            pltpu.make_async_copy(msg_hbm.at[pl.ds(base, C)], msg_v.at[slot], sin[slot]),
            pltpu.make_async_copy(curr_hbm.at[pl.ds(base, C), pl.ds(0, 1)], curr_v.at[slot], sin[slot]),
            pltpu.make_async_copy(alpha_hbm.at[pl.ds(base, C)], alpha_v.at[slot], sin[slot]),
        )

    def out_copy(i, slot):
        base = (wid + i * NW) * C
        return pltpu.make_async_copy(out_v.at[slot], out_hbm.at[pl.ds(base, C)], sout[slot])

    def issue_in(i, slot):
        for c in in_copies(i, slot):
            c.start()

    def wait_in(i, slot):
        for c in in_copies(i, slot):
            c.wait()

    def compute(slot):
        def node_body(n, _):
            av = alpha_v[slot, n, pl.ds(0, DEG)]
            a = [av[k] for k in range(DEG)]
            for j in range(D // L):
                acc = curr_v[slot, n, 0, pl.ds(j * L, L)]
                for k in range(DEG):
                    acc = acc + a[k] * msg_v[slot, n, k, pl.ds(j * L, L)]
                out_v[slot, n, pl.ds(j * L, L)] = acc
            return 0

        lax.fori_loop(0, C, node_body, 0)

    def step(i, slot):
        @pl.when(i + 1 < STEADY)
        def _():
            issue_in(i + 1, slot ^ 1)

        wait_in(i, slot)

        @pl.when(i >= 2)
        def _():
            out_copy(i - 2, slot).wait()

        compute(slot)
        out_copy(i, slot).start()

    issue_in(0, 0)

    def pair_body(t, _):
        i = t * 2
        step(i, 0)
        step(i + 1, 1)
        return 0

    lax.fori_loop(0, (STEADY - 1) // 2, pair_body, 0)
    step(STEADY - 1, (STEADY - 1) % 2)

    out_copy(STEADY - 2, (STEADY - 2) % 2).wait()
    out_copy(STEADY - 1, (STEADY - 1) % 2).wait()

    @pl.when(wid < _EXTRA)
    def _():
        i = STEADY + 0
        issue_in(i, 0)
        wait_in(i, 0)
        compute(0)
        out_copy(i, 0).start()
        out_copy(i, 0).wait()


def kernel(curr_emb, alpha, msg):
    alpha2 = alpha.reshape(N, DEG)
    return _sc_kernel(curr_emb, alpha2, msg)
